# Initial kernel scaffold; baseline (speedup 1.0000x reference)
import jax
import jax.numpy as jnp
from jax.experimental import pallas as pl


def kernel(node_feats, pos, params, edge_index):
    raise NotImplementedError("write your pallas kernel here")


# speedup vs baseline: 1.0071x; 1.0071x over previous
import functools

import jax
import jax.numpy as jnp
from jax.experimental import pallas as pl
from jax.experimental.pallas import tpu as pltpu

F32 = jnp.float32
EB = 2000
CP = 16


def _silu(x):
    return x * (1.0 / (1.0 + jnp.exp(-x)))


def _node0_body(nf_ref, pos_ref, wemb_ref, bemb_ref, wea_ref, web_ref, b1_ref,
                h_ref, a_ref, b_ref, c_ref):
    h = jnp.dot(nf_ref[...], wemb_ref[...], preferred_element_type=F32) + bemb_ref[...]
    h_ref[...] = h
    a_ref[...] = jnp.dot(h, wea_ref[...], preferred_element_type=F32) + b1_ref[...]
    b_ref[...] = jnp.dot(h, web_ref[...], preferred_element_type=F32)
    c_ref[...] = pos_ref[...]


def _node_mid_body(h_ref, coord_ref, sm_ref, st_ref,
                   n1h_ref, n1m_ref, bn1_ref, n2_ref, bn2_ref,
                   wea_ref, web_ref, b1_ref,
                   h_out, coord_out, a_out, b_out):
    n = h_ref.shape[0]
    msum = sm_ref[0] + sm_ref[1]
    stt = st_ref[0] + st_ref[1]
    cnt = jnp.maximum(stt[:, 3:4], 1.0)
    li = jax.lax.broadcasted_iota(jnp.int32, (n, CP), 1)
    upd = jnp.where(li == 3, 0.0, stt / cnt)
    coord_out[...] = coord_ref[...] + upd
    h = h_ref[...]
    pre = (jnp.dot(h, n1h_ref[...], preferred_element_type=F32)
           + jnp.dot(msum, n1m_ref[...], preferred_element_type=F32) + bn1_ref[...])
    h_new = h + jnp.dot(_silu(pre), n2_ref[...], preferred_element_type=F32) + bn2_ref[...]
    h_out[...] = h_new
    a_out[...] = jnp.dot(h_new, wea_ref[...], preferred_element_type=F32) + b1_ref[...]
    b_out[...] = jnp.dot(h_new, web_ref[...], preferred_element_type=F32)


def _edge_body(ga_ref, gb_ref, cr_ref, cc_ref,
               wr_ref, e2_ref, b2_ref, c1_ref, bc1_ref, c2_ref,
               m_ref, tr_ref):
    cd = cr_ref[...] - cc_ref[...]
    radial = jnp.sum(cd * cd, axis=1, keepdims=True)
    g = ga_ref[...] + gb_ref[...] + radial * wr_ref[...]
    t = _silu(g)
    m = _silu(jnp.dot(t, e2_ref[...], preferred_element_type=F32) + b2_ref[...])
    m_ref[...] = m
    u = _silu(jnp.dot(m, c1_ref[...], preferred_element_type=F32) + bc1_ref[...])
    phi = jnp.dot(u, c2_ref[...], preferred_element_type=F32)
    li = jax.lax.broadcasted_iota(jnp.int32, cd.shape, 1)
    tr_ref[...] = jnp.where(li == 3, 1.0, cd * phi)


def _edge_last_body(row_ref, ga_ref, gb_ref, cr_ref, cc_ref,
                    wr_ref, e2_ref, b2_ref, msum0_ref):
    cd = cr_ref[...] - cc_ref[...]
    radial = jnp.sum(cd * cd, axis=1, keepdims=True)
    g = ga_ref[...] + gb_ref[...] + radial * wr_ref[...]
    t = _silu(g)
    m = _silu(jnp.dot(t, e2_ref[...], preferred_element_type=F32) + b2_ref[...])
    maskf = (row_ref[0] == 0).astype(F32)
    part = jnp.dot(maskf, m, preferred_element_type=F32)

    @pl.when(pl.program_id(0) == 0)
    def _():
        msum0_ref[...] = jnp.zeros_like(msum0_ref)

    msum0_ref[...] += part


def _head_body(h8_ref, sm0_ref, n1h_ref, n1m_ref, bn1_ref, n2_ref, bn2_ref,
               wout_ref, bout_ref, wp1_ref, bp1_ref, wp2_ref, bp2_ref, out_ref):
    h0 = h8_ref[0:1, :]
    msum0 = sm0_ref[...]
    pre = (jnp.dot(h0, n1h_ref[...], preferred_element_type=F32)
           + jnp.dot(msum0, n1m_ref[...], preferred_element_type=F32) + bn1_ref[...])
    h4 = h0 + jnp.dot(_silu(pre), n2_ref[...], preferred_element_type=F32) + bn2_ref[...]
    central = jnp.dot(h4, wout_ref[...], preferred_element_type=F32) + bout_ref[...]
    hid = jnp.maximum(jnp.dot(central, wp1_ref[...], preferred_element_type=F32)
                      + bp1_ref[...], 0.0)
    out_ref[...] = jnp.dot(hid, wp2_ref[...], preferred_element_type=F32) + bp2_ref[...]


def _full_spec():
    return pl.BlockSpec(memory_space=pltpu.ANY)


def _node0(node_feats, pos16, p):
    n, hid = node_feats.shape[0], p["emb_in"]["W"].shape[1]
    lay0 = p["layers"][0]
    wea = lay0["e1"]["W"][:hid]
    web = lay0["e1"]["W"][hid:2 * hid]
    b1 = lay0["e1"]["b"].reshape(1, hid)
    out = pl.pallas_call(
        _node0_body,
        out_shape=[jax.ShapeDtypeStruct((n, hid), F32),
                   jax.ShapeDtypeStruct((n, hid), F32),
                   jax.ShapeDtypeStruct((n, hid), F32),
                   jax.ShapeDtypeStruct((n, CP), F32)],
    )(node_feats, pos16, p["emb_in"]["W"], p["emb_in"]["b"].reshape(1, hid),
      wea, web, b1)
    return out


def _node_mid(h, coord, sm, st, lay, nxt):
    n, hid = h.shape
    n1h = lay["n1"]["W"][:hid]
    n1m = lay["n1"]["W"][hid:]
    wea = nxt["e1"]["W"][:hid]
    web = nxt["e1"]["W"][hid:2 * hid]
    b1 = nxt["e1"]["b"].reshape(1, hid)
    return pl.pallas_call(
        _node_mid_body,
        out_shape=[jax.ShapeDtypeStruct((n, hid), F32),
                   jax.ShapeDtypeStruct((n, CP), F32),
                   jax.ShapeDtypeStruct((n, hid), F32),
                   jax.ShapeDtypeStruct((n, hid), F32)],
    )(h, coord, sm, st, n1h, n1m, lay["n1"]["b"].reshape(1, hid),
      lay["n2"]["W"], lay["n2"]["b"].reshape(1, hid), wea, web, b1)


def _edge(ga, gb, cr, cc, lay):
    e, hid = ga.shape
    nblk = e // EB
    wr = lay["e1"]["W"][2 * hid:2 * hid + 1]
    grid = (nblk,)
    bs_h = pl.BlockSpec((EB, hid), lambda i: (i, 0))
    bs_c = pl.BlockSpec((EB, CP), lambda i: (i, 0))
    bs_w = pl.BlockSpec((hid, hid), lambda i: (0, 0))
    bs_b = pl.BlockSpec((1, hid), lambda i: (0, 0))
    return pl.pallas_call(
        _edge_body,
        grid=grid,
        in_specs=[bs_h, bs_h, bs_c, bs_c, bs_b, bs_w, bs_b, bs_w, bs_b,
                  pl.BlockSpec((hid, 1), lambda i: (0, 0))],
        out_specs=[bs_h, bs_c],
        out_shape=[jax.ShapeDtypeStruct((e, hid), F32),
                   jax.ShapeDtypeStruct((e, CP), F32)],
    )(ga, gb, cr, cc, wr, lay["e2"]["W"], lay["e2"]["b"].reshape(1, hid),
      lay["c1"]["W"], lay["c1"]["b"].reshape(1, hid), lay["c2W"])


def _edge_last(row3, ga, gb, cr, cc, lay):
    e, hid = ga.shape
    nblk = e // EB
    wr = lay["e1"]["W"][2 * hid:2 * hid + 1]
    bs_h = pl.BlockSpec((EB, hid), lambda i: (i, 0))
    bs_c = pl.BlockSpec((EB, CP), lambda i: (i, 0))
    bs_w = pl.BlockSpec((hid, hid), lambda i: (0, 0))
    bs_b = pl.BlockSpec((1, hid), lambda i: (0, 0))
    return pl.pallas_call(
        _edge_last_body,
        grid=(nblk,),
        in_specs=[pl.BlockSpec((1, 1, EB), lambda i: (i, 0, 0)),
                  bs_h, bs_h, bs_c, bs_c, bs_b, bs_w, bs_b],
        out_specs=pl.BlockSpec((1, hid), lambda i: (0, 0)),
        out_shape=jax.ShapeDtypeStruct((1, hid), F32),
    )(row3, ga, gb, cr, cc, wr, lay["e2"]["W"], lay["e2"]["b"].reshape(1, hid))


def _head(h8, msum0, lay, p):
    hid = h8.shape[1]
    onf = p["pred2"]["W"].shape[1]
    n1h = lay["n1"]["W"][:hid]
    n1m = lay["n1"]["W"][hid:]
    wp2 = jnp.zeros((hid, hid), F32).at[:, :onf].set(p["pred2"]["W"])
    bp2 = jnp.zeros((1, hid), F32).at[0, :onf].set(p["pred2"]["b"])
    out = pl.pallas_call(
        _head_body,
        out_shape=jax.ShapeDtypeStruct((1, hid), F32),
    )(h8, msum0, n1h, n1m, lay["n1"]["b"].reshape(1, hid),
      lay["n2"]["W"], lay["n2"]["b"].reshape(1, hid),
      p["emb_out"]["W"], p["emb_out"]["b"].reshape(1, hid),
      p["pred1"]["W"], p["pred1"]["b"].reshape(1, hid), wp2, bp2)
    return out[0, :onf]


def _gather(a, b, c, row, col):
    return a[row], b[col], c[row], c[col]


def _scatter(m, tr, row, n):
    sm = jax.ops.segment_sum(m, row, num_segments=n)
    st = jax.ops.segment_sum(tr, row, num_segments=n)
    z128 = jnp.zeros_like(sm)
    zc = jnp.zeros_like(st)
    return jnp.stack([sm, z128]), jnp.stack([st, zc])


def kernel(node_feats, pos, params, edge_index):
    n = node_feats.shape[0]
    e = edge_index.shape[1]
    row = edge_index[0].astype(jnp.int32)
    col = edge_index[1].astype(jnp.int32)
    row3 = row.reshape(e // EB, 1, EB)
    pos16 = jnp.zeros((n, CP), F32).at[:, :3].set(pos)

    layers = params["layers"]
    nl = len(layers)

    h, a, b, c = _node0(node_feats, pos16, params)
    coord = c
    for li in range(nl - 1):
        ga, gb, cr, cc = _gather(a, b, coord, row, col)
        m, tr = _edge(ga, gb, cr, cc, layers[li])
        sm, st = _scatter(m, tr, row, n)
        h, coord, a, b = _node_mid(h, coord, sm, st, layers[li], layers[li + 1])
    ga, gb, cr, cc = _gather(a, b, coord, row, col)
    msum0 = _edge_last(row3, ga, gb, cr, cc, layers[nl - 1])
    logits = _head(h[0:8], msum0, layers[nl - 1], params)
    return logits


# speedup vs baseline: 3.0727x; 3.0511x over previous
import functools

import jax
import jax.numpy as jnp
from jax import lax
from jax.experimental import pallas as pl
from jax.experimental.pallas import tpu as pltpu
from jax.experimental.pallas import tpu_sc as plsc

F32 = jnp.float32
EB = 2000
CP = 16
CH = 128
NW = 32


def _silu(x):
    return x * (1.0 / (1.0 + jnp.exp(-x)))


def _node0_body(nf_ref, pos_ref, wemb_ref, bemb_ref, wea_ref, web_ref, b1_ref,
                h_ref, a_ref, b_ref, c_ref, c8_ref):
    h = jnp.dot(nf_ref[...], wemb_ref[...], preferred_element_type=F32) + bemb_ref[...]
    h_ref[...] = h
    a_ref[...] = jnp.dot(h, wea_ref[...], preferred_element_type=F32) + b1_ref[...]
    b_ref[...] = jnp.dot(h, web_ref[...], preferred_element_type=F32)
    c_ref[...] = pos_ref[...]
    c8_ref[...] = pos_ref[...][:, :8]


def _node_mid_body(h_ref, coord_ref, sm_ref, st_ref,
                   n1h_ref, n1m_ref, bn1_ref, n2_ref, bn2_ref,
                   wea_ref, web_ref, b1_ref,
                   h_out, coord_out, c8_out, a_out, b_out):
    n = h_ref.shape[0]
    msum = sm_ref[0, :n] + sm_ref[1, :n]
    stt = st_ref[0, :n] + st_ref[1, :n]
    cnt = jnp.maximum(stt[:, 3:4], 1.0)
    li = jax.lax.broadcasted_iota(jnp.int32, (n, CP), 1)
    upd = jnp.where(li == 3, 0.0, stt / cnt)
    coord_new = coord_ref[...] + upd
    coord_out[...] = coord_new
    c8_out[...] = coord_new[:, :8]
    h = h_ref[...]
    pre = (jnp.dot(h, n1h_ref[...], preferred_element_type=F32)
           + jnp.dot(msum, n1m_ref[...], preferred_element_type=F32) + bn1_ref[...])
    h_new = h + jnp.dot(_silu(pre), n2_ref[...], preferred_element_type=F32) + bn2_ref[...]
    h_out[...] = h_new
    a_out[...] = jnp.dot(h_new, wea_ref[...], preferred_element_type=F32) + b1_ref[...]
    b_out[...] = jnp.dot(h_new, web_ref[...], preferred_element_type=F32)


def _edge_body(row_ref, ga_ref, gb_ref, cdr_ref,
               wr_ref, e2_ref, b2_ref, c1_ref, bc1_ref, c2_ref,
               m_ref, trs_ref):
    cdr = cdr_ref[...]
    radial = cdr[:, 3:4]
    g = ga_ref[...] + gb_ref[...] + radial * wr_ref[...]
    t = _silu(g)
    m = _silu(jnp.dot(t, e2_ref[...], preferred_element_type=F32) + b2_ref[...])
    m_ref[...] = m
    u = _silu(jnp.dot(m, c1_ref[...], preferred_element_type=F32) + bc1_ref[...])
    phi = jnp.dot(u, c2_ref[...], preferred_element_type=F32)
    li = jax.lax.broadcasted_iota(jnp.int32, cdr.shape, 1)
    tr16 = jnp.where(li == 3, 1.0, cdr * phi)
    eb = cdr.shape[0]
    r = row_ref[...].reshape(eb, 1)
    lg = jax.lax.broadcasted_iota(jnp.int32, (eb, 8 * CP), 1) // CP
    tiled = jnp.concatenate([tr16] * 8, axis=1)
    trs_ref[...] = jnp.where(lg == (r & 7), tiled, 0.0)


def _edge_last_body(row_ref, ga_ref, gb_ref, cdr_ref,
                    wr_ref, e2_ref, b2_ref, msum0_ref):
    radial = cdr_ref[...][:, 3:4]
    g = ga_ref[...] + gb_ref[...] + radial * wr_ref[...]
    t = _silu(g)
    m = _silu(jnp.dot(t, e2_ref[...], preferred_element_type=F32) + b2_ref[...])
    maskf = (row_ref[0] == 0).astype(F32)
    part = jnp.dot(maskf, m, preferred_element_type=F32)

    @pl.when(pl.program_id(0) == 0)
    def _():
        msum0_ref[...] = jnp.zeros_like(msum0_ref)

    msum0_ref[...] += part


def _head_body(h8_ref, sm0_ref, n1h_ref, n1m_ref, bn1_ref, n2_ref, bn2_ref,
               wout_ref, bout_ref, wp1_ref, bp1_ref, wp2_ref, bp2_ref, out_ref):
    h0 = h8_ref[0:1, :]
    msum0 = sm0_ref[...]
    pre = (jnp.dot(h0, n1h_ref[...], preferred_element_type=F32)
           + jnp.dot(msum0, n1m_ref[...], preferred_element_type=F32) + bn1_ref[...])
    h4 = h0 + jnp.dot(_silu(pre), n2_ref[...], preferred_element_type=F32) + bn2_ref[...]
    central = jnp.dot(h4, wout_ref[...], preferred_element_type=F32) + bout_ref[...]
    hid = jnp.maximum(jnp.dot(central, wp1_ref[...], preferred_element_type=F32)
                      + bp1_ref[...], 0.0)
    out_ref[...] = jnp.dot(hid, wp2_ref[...], preferred_element_type=F32) + bp2_ref[...]


def _full_spec():
    return pl.BlockSpec(memory_space=pltpu.ANY)


def _node0(node_feats, pos16, p):
    n, hid = node_feats.shape[0], p["emb_in"]["W"].shape[1]
    lay0 = p["layers"][0]
    wea = lay0["e1"]["W"][:hid]
    web = lay0["e1"]["W"][hid:2 * hid]
    b1 = lay0["e1"]["b"].reshape(1, hid)
    out = pl.pallas_call(
        _node0_body,
        out_shape=[jax.ShapeDtypeStruct((n, hid), F32),
                   jax.ShapeDtypeStruct((n, hid), F32),
                   jax.ShapeDtypeStruct((n, hid), F32),
                   jax.ShapeDtypeStruct((n, CP), F32),
                   jax.ShapeDtypeStruct((n, 8), F32)],
    )(node_feats, pos16, p["emb_in"]["W"], p["emb_in"]["b"].reshape(1, hid),
      wea, web, b1)
    return out


def _node_mid(h, coord, sm, st, lay, nxt):
    n, hid = h.shape
    nb = 2000
    grid = (n // nb,)
    n1h = lay["n1"]["W"][:hid]
    n1m = lay["n1"]["W"][hid:]
    wea = nxt["e1"]["W"][:hid]
    web = nxt["e1"]["W"][hid:2 * hid]
    b1 = nxt["e1"]["b"].reshape(1, hid)
    bs_h = pl.BlockSpec((nb, hid), lambda i: (i, 0))
    bs_c = pl.BlockSpec((nb, CP), lambda i: (i, 0))
    bs_w = pl.BlockSpec((hid, hid), lambda i: (0, 0))
    bs_b = pl.BlockSpec((1, hid), lambda i: (0, 0))
    return pl.pallas_call(
        _node_mid_body,
        grid=grid,
        in_specs=[bs_h, bs_c,
                  pl.BlockSpec((2, nb, hid), lambda i: (0, i, 0)),
                  pl.BlockSpec((2, nb, CP), lambda i: (0, i, 0)),
                  bs_w, bs_w, bs_b, bs_w, bs_b, bs_w, bs_w, bs_b],
        out_specs=[bs_h, bs_c, pl.BlockSpec((nb, 8), lambda i: (i, 0)),
                   bs_h, bs_h],
        out_shape=[jax.ShapeDtypeStruct((n, hid), F32),
                   jax.ShapeDtypeStruct((n, CP), F32),
                   jax.ShapeDtypeStruct((n, 8), F32),
                   jax.ShapeDtypeStruct((n, hid), F32),
                   jax.ShapeDtypeStruct((n, hid), F32)],
    )(h, coord, sm[:, :n], st[:, :n], n1h, n1m, lay["n1"]["b"].reshape(1, hid),
      lay["n2"]["W"], lay["n2"]["b"].reshape(1, hid), wea, web, b1)


def _edge(row3, ga, gb, cdr, lay):
    e, hid = ga.shape
    nblk = e // EB
    wr = lay["e1"]["W"][2 * hid:2 * hid + 1]
    grid = (nblk,)
    bs_h = pl.BlockSpec((EB, hid), lambda i: (i, 0))
    bs_c = pl.BlockSpec((EB, CP), lambda i: (i, 0))
    bs_w = pl.BlockSpec((hid, hid), lambda i: (0, 0))
    bs_b = pl.BlockSpec((1, hid), lambda i: (0, 0))
    return pl.pallas_call(
        _edge_body,
        grid=grid,
        in_specs=[pl.BlockSpec((1, 1, EB), lambda i: (i, 0, 0)),
                  bs_h, bs_h, bs_c, bs_b, bs_w, bs_b, bs_w, bs_b,
                  pl.BlockSpec((hid, 1), lambda i: (0, 0))],
        out_specs=[bs_h, bs_h],
        out_shape=[jax.ShapeDtypeStruct((e, hid), F32),
                   jax.ShapeDtypeStruct((e, hid), F32)],
    )(row3, ga, gb, cdr, wr, lay["e2"]["W"], lay["e2"]["b"].reshape(1, hid),
      lay["c1"]["W"], lay["c1"]["b"].reshape(1, hid), lay["c2W"])


def _edge_last(row3, ga, gb, cdr, lay):
    e, hid = ga.shape
    nblk = e // EB
    wr = lay["e1"]["W"][2 * hid:2 * hid + 1]
    bs_h = pl.BlockSpec((EB, hid), lambda i: (i, 0))
    bs_c = pl.BlockSpec((EB, CP), lambda i: (i, 0))
    bs_w = pl.BlockSpec((hid, hid), lambda i: (0, 0))
    bs_b = pl.BlockSpec((1, hid), lambda i: (0, 0))
    return pl.pallas_call(
        _edge_last_body,
        grid=(nblk,),
        in_specs=[pl.BlockSpec((1, 1, EB), lambda i: (i, 0, 0)),
                  bs_h, bs_h, bs_c, bs_b, bs_w, bs_b],
        out_specs=pl.BlockSpec((1, hid), lambda i: (0, 0)),
        out_shape=jax.ShapeDtypeStruct((1, hid), F32),
    )(row3, ga, gb, cdr, wr, lay["e2"]["W"], lay["e2"]["b"].reshape(1, hid))


def _head(h8, msum0, lay, p):
    hid = h8.shape[1]
    onf = p["pred2"]["W"].shape[1]
    n1h = lay["n1"]["W"][:hid]
    n1m = lay["n1"]["W"][hid:]
    wp2 = jnp.zeros((hid, hid), F32).at[:, :onf].set(p["pred2"]["W"])
    bp2 = jnp.zeros((1, hid), F32).at[0, :onf].set(p["pred2"]["b"])
    out = pl.pallas_call(
        _head_body,
        out_shape=jax.ShapeDtypeStruct((1, hid), F32),
    )(h8, msum0, n1h, n1m, lay["n1"]["b"].reshape(1, hid),
      lay["n2"]["W"], lay["n2"]["b"].reshape(1, hid),
      p["emb_out"]["W"], p["emb_out"]["b"].reshape(1, hid),
      p["pred1"]["W"], p["pred1"]["b"].reshape(1, hid), wp2, bp2)
    return out[0, :onf]


def _gather(a, b, c8, row, col):
    n, hid = a.shape
    e = row.shape[0]
    nchunk = e // CH
    niter = (nchunk + NW - 1) // NW
    mesh = plsc.VectorSubcoreMesh(core_axis_name="c", subcore_axis_name="s")

    @functools.partial(
        pl.kernel, mesh=mesh,
        out_type=[jax.ShapeDtypeStruct((e, hid), F32),
                  jax.ShapeDtypeStruct((e, hid), F32),
                  jax.ShapeDtypeStruct((e * CP,), F32)],
        scratch_types=[pltpu.VMEM((CH,), jnp.int32),
                       pltpu.VMEM((CH,), jnp.int32),
                       pltpu.VMEM((CH, hid), F32),
                       pltpu.VMEM((CH, hid), F32),
                       pltpu.VMEM((CH * CP,), F32),
                       pltpu.VMEM((n * 8,), F32),
                       pltpu.SemaphoreType.DMA],
        compiler_params=pltpu.CompilerParams(needs_layout_passes=False),
    )
    def gk(a_hbm, b_hbm, c_hbm, row_hbm, col_hbm, oga, ogb, ocdr,
           idxr, idxc, bufa, bufb, bufcdr, ctab, sem):
        wid = lax.axis_index("s") * 2 + lax.axis_index("c")
        pltpu.sync_copy(c_hbm, ctab)
        for z in range(CH * CP // 16):
            bufcdr[pl.ds(z * 16, 16)] = jnp.zeros((16,), F32)

        def body(i, carry):
            ch = wid + i * NW

            @pl.when(ch < nchunk)
            def _():
                base = ch * CH
                pltpu.sync_copy(row_hbm.at[pl.ds(base, CH)], idxr)
                pltpu.sync_copy(col_hbm.at[pl.ds(base, CH)], idxc)
                ca = pltpu.async_copy(a_hbm.at[idxr], bufa, sem)
                cb = pltpu.async_copy(b_hbm.at[idxc], bufb, sem)
                for s in range(CH // 16):
                    rv = idxr[pl.ds(s * 16, 16)] * 8
                    cv = idxc[pl.ds(s * 16, 16)] * 8
                    loc = (jax.lax.iota(jnp.int32, 16) + (s * 16)) * CP
                    rad = jnp.zeros((16,), F32)
                    for k in range(3):
                        kk = jnp.full((16,), k, jnp.int32)
                        d = (plsc.load_gather(ctab, [rv + kk])
                             - plsc.load_gather(ctab, [cv + kk]))
                        plsc.store_scatter(bufcdr, [loc + kk], d)
                        rad = rad + d * d
                    plsc.store_scatter(bufcdr, [loc + jnp.full((16,), 3, jnp.int32)], rad)
                ca.wait()
                cb.wait()
                pltpu.sync_copy(bufa, oga.at[pl.ds(base, CH)])
                pltpu.sync_copy(bufb, ogb.at[pl.ds(base, CH)])
                pltpu.sync_copy(bufcdr, ocdr.at[pl.ds(base * CP, CH * CP)])
            return carry

        lax.fori_loop(0, niter, body, 0)

    return gk(a, b, c8, row, col)


def _scatter(m, trs, row, n):
    e, hid = m.shape
    nchunk = e // CH
    niter = (nchunk + NW - 1) // NW
    npad = ((n + 127) // 128) * 128
    rows_per_tile = npad // 16
    npad2 = ((n + 1023) // 1024) * 128
    rows2_per_tile = npad2 // 16
    z128 = jnp.zeros((CH, hid), F32)
    mesh = plsc.VectorSubcoreMesh(core_axis_name="c", subcore_axis_name="s")

    @functools.partial(
        pl.kernel, mesh=mesh,
        out_type=[jax.ShapeDtypeStruct((2 * npad, hid), F32),
                  jax.ShapeDtypeStruct((2 * npad2, hid), F32)],
        scratch_types=[pltpu.VMEM((CH,), jnp.int32),
                       pltpu.VMEM((CH, hid), F32),
                       pltpu.VMEM((CH, hid), F32),
                       pltpu.VMEM_SHARED((npad, hid), F32),
                       pltpu.VMEM_SHARED((npad2, hid), F32),
                       pltpu.SemaphoreType.DMA],
    )
    def sk(m_hbm, t_hbm, row_hbm, z128_hbm, osm, ost,
           idx, bufm, buft, accm, acct, sem):
        cid = lax.axis_index("c")
        sid = lax.axis_index("s")
        wid = sid * 2 + cid
        rbase = sid * rows_per_tile
        r2base = sid * rows2_per_tile
        pltpu.sync_copy(z128_hbm, bufm)
        nfull = rows_per_tile // CH
        rem = rows_per_tile - nfull * CH
        for j in range(nfull):
            pltpu.sync_copy(bufm, accm.at[pl.ds(rbase + j * CH, CH)])
        if rem:
            pltpu.sync_copy(bufm.at[pl.ds(0, rem)],
                            accm.at[pl.ds(rbase + nfull * CH, rem)])
        pltpu.sync_copy(bufm.at[pl.ds(0, rows2_per_tile)],
                        acct.at[pl.ds(r2base, rows2_per_tile)])
        plsc.subcore_barrier()

        def body(i, carry):
            ch = wid + i * NW

            @pl.when(ch < nchunk)
            def _():
                base = ch * CH
                pltpu.sync_copy(row_hbm.at[pl.ds(base, CH)], idx)
                pltpu.sync_copy(m_hbm.at[pl.ds(base, CH)], bufm)
                pltpu.sync_copy(t_hbm.at[pl.ds(base, CH)], buft)
                pltpu.sync_copy(bufm, accm.at[idx], add=True)
                for s in range(CH // 16):
                    v = idx[pl.ds(s * 16, 16)]
                    idx[pl.ds(s * 16, 16)] = jax.lax.shift_right_logical(v, 3)
                pltpu.sync_copy(buft, acct.at[idx], add=True)
            return carry

        lax.fori_loop(0, niter, body, 0)
        plsc.subcore_barrier()
        obase = cid * npad + rbase
        for j in range(nfull):
            pltpu.sync_copy(accm.at[pl.ds(rbase + j * CH, CH)], bufm)
            pltpu.sync_copy(bufm, osm.at[pl.ds(obase + j * CH, CH)])
        if rem:
            pltpu.sync_copy(accm.at[pl.ds(rbase + nfull * CH, rem)],
                            bufm.at[pl.ds(0, rem)])
            pltpu.sync_copy(bufm.at[pl.ds(0, rem)],
                            osm.at[pl.ds(obase + nfull * CH, rem)])
        pltpu.sync_copy(acct.at[pl.ds(r2base, rows2_per_tile)],
                        buft.at[pl.ds(0, rows2_per_tile)])
        pltpu.sync_copy(buft.at[pl.ds(0, rows2_per_tile)],
                        ost.at[pl.ds(cid * npad2 + r2base, rows2_per_tile)])

    sm2, st2 = sk(m, trs, row, z128)
    return sm2.reshape(2, npad, hid), st2.reshape(2, 8 * npad2, CP)


def kernel(node_feats, pos, params, edge_index):
    n = node_feats.shape[0]
    e = edge_index.shape[1]
    row = edge_index[0].astype(jnp.int32)
    col = edge_index[1].astype(jnp.int32)
    row3 = row.reshape(e // EB, 1, EB)
    pos16 = jnp.zeros((n, CP), F32).at[:, :3].set(pos)

    layers = params["layers"]
    nl = len(layers)

    h, a, b, coord, c8 = _node0(node_feats, pos16, params)
    for li in range(nl - 1):
        ga, gb, cdrf = _gather(a, b, c8.reshape(-1), row, col)
        m, trs = _edge(row3, ga, gb, cdrf.reshape(e, CP), layers[li])
        sm, st = _scatter(m, trs, row, n)
        h, coord, c8, a, b = _node_mid(h, coord, sm, st, layers[li], layers[li + 1])
    ga, gb, cdrf = _gather(a, b, c8.reshape(-1), row, col)
    msum0 = _edge_last(row3, ga, gb, cdrf.reshape(e, CP), layers[nl - 1])
    logits = _head(h[0:8], msum0, layers[nl - 1], params)
    return logits
